# Initial kernel scaffold; baseline (speedup 1.0000x reference)
import jax
import jax.numpy as jnp
from jax.experimental import pallas as pl


def kernel(x, edge_index, edge_attr, batch, Wp, bp, Wf, bf, Ws, bs, g_in, b_in, g_out, b_out, W1, b1, W2, b2):
    raise NotImplementedError("write your pallas kernel here")


# speedup vs baseline: 1.8309x; 1.8309x over previous
import functools

import jax
import jax.numpy as jnp
from jax import lax
from jax.experimental import pallas as pl
from jax.experimental.pallas import tpu as pltpu
from jax.experimental.pallas import tpu_sc as plsc

N = 10000
E = 320000
H = 128
L = 3
DE = 13
DEP = 16
G = 64

NC = 2
NS = 16
NW = NC * NS
EPT = E // NW
K = 80
CHUNKS = EPT // K
NP = 10240
ROWS_PT = NP // NS

_HI = jax.lax.Precision.HIGHEST


def _lin0_body(x_ref, wp_ref, bp_ref, h_ref):
    h_ref[...] = jnp.dot(x_ref[...], wp_ref[...]) + bp_ref[...]


def _proj_body(h_ref, wd_ref, wsr_ref, pd_ref, ps_ref):
    h = h_ref[...]
    pd_ref[...] = jnp.dot(h, wd_ref[...])
    ps_ref[...] = jnp.dot(h, wsr_ref[...])


def _msg_body(td_ref, ts_ref, ea_ref, we_ref, bc_ref, m_ref):
    u = (td_ref[...] + ts_ref[...]
         + jnp.dot(ea_ref[...], we_ref[...]) + bc_ref[...])
    uf = u[:, :H]
    us = u[:, H:]
    sg = 1.0 / (1.0 + jnp.exp(-uf))
    sp = jnp.maximum(us, 0.0) + jnp.log1p(jnp.exp(-jnp.abs(us)))
    m_ref[...] = sg * sp


def _bn(v, g, b):
    mu = jnp.mean(v, axis=0, keepdims=True)
    var = jnp.mean((v - mu) * (v - mu), axis=0, keepdims=True)
    return g * (v - mu) / jnp.sqrt(var + 1e-5) + b


def _update_body(h_ref, p_ref, gi_ref, bi_ref, go_ref, bo_ref, h2_ref):
    agg = _bn(p_ref[0, :N] + p_ref[1, :N], gi_ref[...], bi_ref[...])
    h2_ref[...] = jnp.maximum(_bn(h_ref[...] + agg, go_ref[...], bo_ref[...]), 0.0)


def _final_body(h_ref, p_ref, gi_ref, bi_ref, go_ref, bo_ref, batch_ref,
                w1_ref, b1_ref, w2_ref, b2_ref, o_ref):
    agg = _bn(p_ref[0, :N] + p_ref[1, :N], gi_ref[...], bi_ref[...])
    h = jnp.maximum(_bn(h_ref[...] + agg, go_ref[...], bo_ref[...]), 0.0)
    seg = lax.broadcasted_iota(jnp.int32, (G, N), 0)
    oht = (seg == batch_ref[...]).astype(jnp.float32)
    sums = jnp.dot(oht, h, precision=_HI)
    cnt = jnp.sum(oht, axis=1, keepdims=True)
    pooled = sums / jnp.maximum(cnt, 1.0)
    o1 = jnp.maximum(jnp.dot(pooled, w1_ref[...]) + b1_ref[...], 0.0)
    o_ref[...] = jnp.dot(o1, w2_ref[...]) + b2_ref[...]


def _gather_body(dst_ref, src_ref, pd_ref, ps_ref, td_ref, ts_ref,
                 idxd, idxs, bufd, bufs, semd, sems):
    c = lax.axis_index("c")
    s = lax.axis_index("s")
    base = (s * NC + c) * EPT

    def body(i, carry):
        off = pl.multiple_of(base + i * K, 8)
        pltpu.sync_copy(dst_ref.at[pl.ds(off, K)], idxd)
        pltpu.sync_copy(src_ref.at[pl.ds(off, K)], idxs)
        cpd = pltpu.async_copy(pd_ref.at[idxd], bufd, semd)
        cps = pltpu.async_copy(ps_ref.at[idxs], bufs, sems)
        cpd.wait()
        cps.wait()
        pltpu.sync_copy(bufd, td_ref.at[pl.ds(off, K)])
        pltpu.sync_copy(bufs, ts_ref.at[pl.ds(off, K)])
        return carry

    lax.fori_loop(0, CHUNKS, body, 0)


def _scatter_body(dst_ref, m_ref, z_ref, out_ref, idxb, mb, agg_sh):
    c = lax.axis_index("c")
    s = lax.axis_index("s")
    pltpu.sync_copy(z_ref.at[pl.ds(s * ROWS_PT, ROWS_PT)],
                    agg_sh.at[pl.ds(s * ROWS_PT, ROWS_PT)])
    plsc.subcore_barrier()
    base = c * (E // NC) + s * EPT

    def body(i, carry):
        off = pl.multiple_of(base + i * K, 8)
        pltpu.sync_copy(dst_ref.at[pl.ds(off, K)], idxb)
        pltpu.sync_copy(m_ref.at[pl.ds(off, K)], mb)
        pltpu.sync_copy(mb, agg_sh.at[idxb], add=True)
        return carry

    lax.fori_loop(0, CHUNKS, body, 0)
    plsc.subcore_barrier()
    pltpu.sync_copy(agg_sh.at[pl.ds(s * ROWS_PT, ROWS_PT)],
                    out_ref.at[c, pl.ds(s * ROWS_PT, ROWS_PT)])


@functools.lru_cache(maxsize=None)
def _sc_calls():
    mesh = plsc.VectorSubcoreMesh(core_axis_name="c", subcore_axis_name="s",
                                  num_cores=NC, num_subcores=NS)
    gather = pl.kernel(
        _gather_body,
        out_type=(jax.ShapeDtypeStruct((E, 2 * H), jnp.float32),
                  jax.ShapeDtypeStruct((E, 2 * H), jnp.float32)),
        mesh=mesh,
        scratch_types=[
            pltpu.VMEM((K,), jnp.int32),
            pltpu.VMEM((K,), jnp.int32),
            pltpu.VMEM((K, 2 * H), jnp.float32),
            pltpu.VMEM((K, 2 * H), jnp.float32),
            pltpu.SemaphoreType.DMA,
            pltpu.SemaphoreType.DMA,
        ],
    )
    scatter = pl.kernel(
        _scatter_body,
        out_type=jax.ShapeDtypeStruct((NC, NP, H), jnp.float32),
        mesh=mesh,
        scratch_types=[
            pltpu.VMEM((K,), jnp.int32),
            pltpu.VMEM((K, H), jnp.float32),
            pltpu.VMEM_SHARED((NP, H), jnp.float32),
        ],
    )
    return gather, scatter


def _full(shape):
    return pl.BlockSpec(shape, lambda: tuple(0 for _ in shape))


_lin0_call = pl.pallas_call(
    _lin0_body,
    out_shape=jax.ShapeDtypeStruct((N, H), jnp.float32),
)

_proj_call = pl.pallas_call(
    _proj_body,
    out_shape=(jax.ShapeDtypeStruct((N, 2 * H), jnp.float32),
               jax.ShapeDtypeStruct((N, 2 * H), jnp.float32)),
)

_BE = 1000

_msg_call = pl.pallas_call(
    _msg_body,
    grid=(E // _BE,),
    in_specs=[
        pl.BlockSpec((_BE, 2 * H), lambda i: (i, 0)),
        pl.BlockSpec((_BE, 2 * H), lambda i: (i, 0)),
        pl.BlockSpec((_BE, DEP), lambda i: (i, 0)),
        pl.BlockSpec((DEP, 2 * H), lambda i: (0, 0)),
        pl.BlockSpec((1, 2 * H), lambda i: (0, 0)),
    ],
    out_specs=pl.BlockSpec((_BE, H), lambda i: (i, 0)),
    out_shape=jax.ShapeDtypeStruct((E, H), jnp.float32),
)

_update_call = pl.pallas_call(
    _update_body,
    out_shape=jax.ShapeDtypeStruct((N, H), jnp.float32),
)

_final_call = pl.pallas_call(
    _final_body,
    out_shape=jax.ShapeDtypeStruct((G, 1), jnp.float32),
)


def kernel(x, edge_index, edge_attr, batch, Wp, bp, Wf, bf, Ws, bs,
           g_in, b_in, g_out, b_out, W1, b1, W2, b2):
    src = edge_index[0].astype(jnp.int32)
    dst = edge_index[1].astype(jnp.int32)
    ea = jnp.pad(edge_attr, ((0, 0), (0, DEP - DE)))
    zeros = jnp.zeros((NP, H), jnp.float32)
    batch_row = batch.astype(jnp.int32).reshape(1, N)

    wd = [jnp.concatenate([Wf[l][:H], Ws[l][:H]], axis=1) for l in range(L)]
    wsr = [jnp.concatenate([Wf[l][H:2 * H], Ws[l][H:2 * H]], axis=1) for l in range(L)]
    we = [jnp.pad(jnp.concatenate([Wf[l][2 * H:], Ws[l][2 * H:]], axis=1),
                  ((0, DEP - DE), (0, 0))) for l in range(L)]
    bc = [jnp.concatenate([bf[l], bs[l]]).reshape(1, 2 * H) for l in range(L)]

    h = _lin0_call(x, Wp, bp.reshape(1, H))
    _gather_call, _scatter_call = _sc_calls()

    o = None
    for l in range(L):
        pd, ps = _proj_call(h, wd[l], wsr[l])
        td, ts = _gather_call(dst, src, pd, ps)
        m = _msg_call(td, ts, ea, we[l], bc[l])
        parts = _scatter_call(dst, m, zeros)
        gi = g_in[l].reshape(1, H)
        bi = b_in[l].reshape(1, H)
        go = g_out[l].reshape(1, H)
        bo = b_out[l].reshape(1, H)
        if l < L - 1:
            h = _update_call(h, parts, gi, bi, go, bo)
        else:
            o = _final_call(h, parts, gi, bi, go, bo, batch_row,
                            W1, b1.reshape(1, H // 2), W2, b2.reshape(1, 1))
    return o
